# baseline (speedup 1.0000x reference)
import functools

import jax
import jax.numpy as jnp
from jax import lax
from jax.experimental import pallas as pl
from jax.experimental.pallas import tpu as pltpu
from jax.experimental.pallas import tpu_sc as plsc

_ROWS = 4096
_COLS = 50
_EMBED = 64
_VOCAB = 1000
_NW = 32
_IW = _ROWS // _NW
_NBUF = 2
_L = 16

_mesh = plsc.VectorSubcoreMesh(core_axis_name="c", subcore_axis_name="s")


@functools.partial(
    pl.kernel,
    mesh=_mesh,
    compiler_params=pltpu.CompilerParams(needs_layout_passes=False),
    out_type=jax.ShapeDtypeStruct((_COLS, _EMBED, _ROWS), jnp.float32),
    scratch_types=[
        pltpu.VMEM((_EMBED * _VOCAB,), jnp.float32),
        pltpu.VMEM((_COLS, _IW), jnp.int32),
        pltpu.VMEM((_NBUF, _EMBED, _IW), jnp.float32),
        pltpu.SemaphoreType.DMA((_NBUF,)),
    ],
)
def _sc_lookup(xt_hbm, enc_hbm, out_hbm, tbl_v, idx_v, stage_v, ssem):
    wid = lax.axis_index("s") * 2 + lax.axis_index("c")
    i0 = wid * _IW

    pltpu.sync_copy(enc_hbm, tbl_v)
    pltpu.sync_copy(xt_hbm.at[:, pl.ds(i0, _IW)], idx_v)

    def plane(j, carry):
        b = j % _NBUF
        dst = out_hbm.at[j, :, pl.ds(i0, _IW)]

        @pl.when(j >= _NBUF)
        def _():
            pltpu.make_async_copy(stage_v.at[b], dst, ssem.at[b]).wait()

        for g in range(_IW // _L):
            ivec = idx_v[j, pl.ds(g * _L, _L)]
            for e in range(_EMBED):
                vals = plsc.load_gather(tbl_v, [ivec + e * _VOCAB])
                stage_v[b, e, pl.ds(g * _L, _L)] = vals
        pltpu.async_copy(stage_v.at[b], dst, ssem.at[b])
        return carry

    lax.fori_loop(0, _COLS, plane, 0)
    for j in range(_COLS - _NBUF, _COLS):
        pltpu.make_async_copy(
            stage_v.at[j % _NBUF], out_hbm.at[j, :, pl.ds(i0, _IW)],
            ssem.at[j % _NBUF]).wait()


def kernel(x, encoding):
    xt = x.T
    enc_t = encoding.T.reshape(_EMBED * _VOCAB)
    out2 = _sc_lookup(xt, enc_t)
    return out2.transpose(2, 0, 1)


# speedup vs baseline: 1.0248x; 1.0248x over previous
import functools

import jax
import jax.numpy as jnp
from jax import lax
from jax.experimental import pallas as pl
from jax.experimental.pallas import tpu as pltpu
from jax.experimental.pallas import tpu_sc as plsc

_ROWS = 4096
_COLS = 50
_EMBED = 64
_VOCAB = 1000
_NW = 32
_IW = _ROWS // _NW
_NBUF = 2
_L = 16

_mesh = plsc.VectorSubcoreMesh(core_axis_name="c", subcore_axis_name="s")


@functools.partial(
    pl.kernel,
    mesh=_mesh,
    compiler_params=pltpu.CompilerParams(needs_layout_passes=False),
    out_type=jax.ShapeDtypeStruct((_COLS, _EMBED, _ROWS), jnp.float32),
    scratch_types=[
        pltpu.VMEM((_EMBED * _VOCAB,), jnp.float32),
        pltpu.VMEM((_COLS, _IW), jnp.int32),
        pltpu.VMEM((_NBUF, _EMBED, _IW), jnp.float32),
        pltpu.SemaphoreType.DMA((_NBUF,)),
    ],
)
def _sc_lookup(xt_hbm, enc_hbm, out_hbm, tbl_v, idx_v, stage_v, ssem):
    wid = lax.axis_index("s") * 2 + lax.axis_index("c")
    i0 = wid * _IW

    pltpu.sync_copy(enc_hbm, tbl_v)
    pltpu.sync_copy(xt_hbm.at[:, pl.ds(i0, _IW)], idx_v)

    def plane(j, carry):
        b = j % _NBUF
        dst = out_hbm.at[j, :, pl.ds(i0, _IW)]

        @pl.when(j >= _NBUF)
        def _():
            pltpu.make_async_copy(stage_v.at[b], dst, ssem.at[b]).wait()

        ivecs = [idx_v[j, pl.ds(g * _L, _L)] for g in range(_IW // _L)]
        for e in range(_EMBED):
            for g in range(_IW // _L):
                vals = plsc.load_gather(tbl_v, [ivecs[g] + e * _VOCAB])
                stage_v[b, e, pl.ds(g * _L, _L)] = vals
        pltpu.async_copy(stage_v.at[b], dst, ssem.at[b])
        return carry

    lax.fori_loop(0, _COLS, plane, 0)
    for j in range(_COLS - _NBUF, _COLS):
        pltpu.make_async_copy(
            stage_v.at[j % _NBUF], out_hbm.at[j, :, pl.ds(i0, _IW)],
            ssem.at[j % _NBUF]).wait()


def kernel(x, encoding):
    xt = x.T
    enc_t = encoding.T.reshape(_EMBED * _VOCAB)
    out2 = _sc_lookup(xt, enc_t)
    return out2.transpose(2, 0, 1)
